# Initial kernel scaffold; baseline (speedup 1.0000x reference)
import jax
import jax.numpy as jnp
from jax.experimental import pallas as pl


def kernel(x, edge_index, batch, W1, b1, W2, b2, W3, b3, Wfc, bfc):
    raise NotImplementedError("write your pallas kernel here")


# speedup vs baseline: 10.5826x; 10.5826x over previous
import functools

import jax
import jax.numpy as jnp
from jax import lax
from jax.experimental import pallas as pl
from jax.experimental.pallas import tpu as pltpu
from jax.experimental.pallas import tpu_sc as plsc

N = 10000
E = 320000
H = 128
G = 64
OUT = 128

NC = 2
NS = 16
NW = NC * NS
CHUNK = 128
CH = 79
E_PAD = NW * CH * CHUNK
N_PAD = 10112
ROWS_PW = N_PAD // NS

_mesh = plsc.VectorSubcoreMesh(core_axis_name="c", subcore_axis_name="s")


@functools.partial(
    pl.kernel,
    mesh=_mesh,
    out_type=jax.ShapeDtypeStruct((NC, N_PAD, 16), jnp.float32),
    scratch_types=[
        pltpu.VMEM((CH, CHUNK), jnp.int32),
        pltpu.VMEM((CHUNK, 16), jnp.float32),
        pltpu.VMEM_SHARED((N_PAD, 16), jnp.float32),
    ],
)
def _deg_sc(dst_hbm, zero_hbm, out_hbm, dst_v, ones_v, acc_sh):
    c = lax.axis_index("c")
    s = lax.axis_index("s")
    wid = s * NC + c

    @pl.loop(0, CHUNK)
    def _(r):
        ones_v[r, :] = jnp.full((16,), 1.0, jnp.float32)

    pltpu.sync_copy(zero_hbm.at[pl.ds(s * ROWS_PW, ROWS_PW)],
                    acc_sh.at[pl.ds(s * ROWS_PW, ROWS_PW)])
    pltpu.sync_copy(dst_hbm.at[wid], dst_v)
    plsc.subcore_barrier()

    @pl.loop(0, CH)
    def _(j):
        pltpu.sync_copy(ones_v, acc_sh.at[dst_v.at[j]], add=True)

    plsc.subcore_barrier()
    pltpu.sync_copy(acc_sh.at[pl.ds(s * ROWS_PW, ROWS_PW)],
                    out_hbm.at[c, pl.ds(s * ROWS_PW, ROWS_PW)])


@functools.partial(
    pl.kernel,
    mesh=_mesh,
    out_type=jax.ShapeDtypeStruct((NC, N_PAD, H), jnp.float32),
    scratch_types=[
        pltpu.VMEM((CH, CHUNK), jnp.int32),
        pltpu.VMEM((CH, CHUNK), jnp.int32),
        pltpu.VMEM((CHUNK, H), jnp.float32),
        pltpu.VMEM_SHARED((N_PAD, H), jnp.float32),
        pltpu.SemaphoreType.DMA,
    ],
)
def _prop_sc(hp_hbm, src_hbm, dst_hbm, zero_hbm, out_hbm,
             src_v, dst_v, rows_v, acc_sh, sem):
    c = lax.axis_index("c")
    s = lax.axis_index("s")
    wid = s * NC + c

    pltpu.sync_copy(zero_hbm.at[pl.ds(s * ROWS_PW, ROWS_PW)],
                    acc_sh.at[pl.ds(s * ROWS_PW, ROWS_PW)])
    pltpu.sync_copy(src_hbm.at[wid], src_v)
    pltpu.sync_copy(dst_hbm.at[wid], dst_v)
    plsc.subcore_barrier()

    @pl.loop(0, CH)
    def _(j):
        pltpu.async_copy(hp_hbm.at[src_v.at[j]], rows_v, sem).wait()
        pltpu.sync_copy(rows_v, acc_sh.at[dst_v.at[j]], add=True)

    plsc.subcore_barrier()
    pltpu.sync_copy(acc_sh.at[pl.ds(s * ROWS_PW, ROWS_PW)],
                    out_hbm.at[c, pl.ds(s * ROWS_PW, ROWS_PW)])


def _pre_body(x_ref, w_ref, deg_ref, hp_ref, dis_ref):
    deg = deg_ref[0, 0:N, 0:1] + deg_ref[1, 0:N, 0:1] + 1.0
    dis = lax.rsqrt(deg)
    dis_ref[...] = dis
    h = jnp.dot(x_ref[...], w_ref[...], preferred_element_type=jnp.float32)
    hp_ref[...] = h * dis


def _pre_tc(x, w, deg):
    return pl.pallas_call(
        _pre_body,
        out_shape=(jax.ShapeDtypeStruct((N, H), jnp.float32),
                   jax.ShapeDtypeStruct((N, 1), jnp.float32)),
    )(x, w, deg)


def _mid_body(a_ref, hp_ref, dis_ref, b_ref, w_ref, out_ref):
    dis = dis_ref[...]
    a = a_ref[0, 0:N, :] + a_ref[1, 0:N, :] + hp_ref[...]
    g = a * dis + b_ref[...]
    r = jnp.maximum(g, 0.0)
    out_ref[...] = jnp.dot(
        r, w_ref[...], preferred_element_type=jnp.float32) * dis


def _mid_tc(a, hp, dis, b, w):
    return pl.pallas_call(
        _mid_body,
        out_shape=jax.ShapeDtypeStruct((N, H), jnp.float32),
    )(a, hp, dis, b, w)


def _fin_body(a_ref, hp_ref, dis_ref, b_ref, batch_ref, wfc_ref, bfc_ref,
              out_ref):
    h3 = (a_ref[0, 0:N, :] + a_ref[1, 0:N, :] + hp_ref[...]) * dis_ref[...] \
        + b_ref[...]
    grp = lax.broadcasted_iota(jnp.int32, (G, N), 0)
    onehot = (batch_ref[...] == grp).astype(jnp.float32)
    sums = jnp.dot(onehot, h3, preferred_element_type=jnp.float32)
    cnt = jnp.sum(onehot, axis=1, keepdims=True)
    pooled = sums / jnp.maximum(cnt, 1.0)
    out_ref[...] = jnp.dot(
        pooled, wfc_ref[...], preferred_element_type=jnp.float32) + bfc_ref[...]


def _fin_tc(a, hp, dis, b, batch2d, wfc, bfc):
    return pl.pallas_call(
        _fin_body,
        out_shape=jax.ShapeDtypeStruct((G, OUT), jnp.float32),
    )(a, hp, dis, b, batch2d, wfc, bfc)


def kernel(x, edge_index, batch, W1, b1, W2, b2, W3, b3, Wfc, bfc):
    src = edge_index[0]
    dst = edge_index[1]
    pad = E_PAD - E
    src3 = jnp.concatenate(
        [src, jnp.zeros((pad,), jnp.int32)]).reshape(NW, CH, CHUNK)
    dst3 = jnp.concatenate(
        [dst, jnp.full((pad,), N, jnp.int32)]).reshape(NW, CH, CHUNK)
    zeros_h = jnp.zeros((N_PAD, H), jnp.float32)
    zeros_d = jnp.zeros((N_PAD, 16), jnp.float32)

    deg = _deg_sc(dst3, zeros_d)
    hp1, dis = _pre_tc(x, W1, deg)
    a1 = _prop_sc(hp1, src3, dst3, zeros_h)
    hp2 = _mid_tc(a1, hp1, dis, b1.reshape(1, H), W2)
    a2 = _prop_sc(hp2, src3, dst3, zeros_h)
    hp3 = _mid_tc(a2, hp2, dis, b2.reshape(1, H), W3)
    a3 = _prop_sc(hp3, src3, dst3, zeros_h)
    return _fin_tc(a3, hp3, dis, b3.reshape(1, H), batch.reshape(1, N),
                   Wfc, bfc.reshape(1, OUT))
